# Initial kernel scaffold; baseline (speedup 1.0000x reference)
import functools

import jax
import jax.numpy as jnp
from jax import lax
from jax.experimental import pallas as pl
from jax.experimental.pallas import tpu as pltpu
from jax.experimental.pallas import tpu_sc as plsc

_NC = 2
_NS = 16


def _swish(x):
    return x * jax.nn.sigmoid(x)


def _t0_body(state_ref, emb_ref, w_ref, b_ref, out_ref):
    st = state_ref[...]
    e0 = emb_ref[0:1, :]
    e1 = emb_ref[1:2, :]
    h = jnp.where(st == 0, e0, e1)
    x = jnp.dot(h, w_ref[...], preferred_element_type=jnp.float32) + b_ref[...]
    out_ref[...] = _swish(x)


def _tmid_body(hfp_ref, agg_ref, w_ref, b_ref, wl_ref, hf_out, c_out, z_out):
    n = hfp_ref.shape[0]
    h = hfp_ref[...] + agg_ref[0:n, :] + agg_ref[n:2 * n, :]
    x = jnp.dot(h, w_ref[...], preferred_element_type=jnp.float32) + b_ref[...]
    hf_out[...] = _swish(x)
    c_out[...] = jnp.dot(h, wl_ref[...], preferred_element_type=jnp.float32)
    z_out[...] = jnp.sum(h, axis=0, keepdims=True)


def _t3_body(hf3_ref, agg_ref, c1_ref, c2_ref, wl3_ref, bl_ref, z1_ref,
             z2_ref, wv_ref, bv_ref, logits_out, values_out):
    n = hf3_ref.shape[0]
    hh = wv_ref.shape[0] // 3
    h = hf3_ref[...] + agg_ref[0:n, :] + agg_ref[n:2 * n, :]
    c3 = jnp.dot(h, wl3_ref[...], preferred_element_type=jnp.float32)
    t = c1_ref[...] + c2_ref[...] + c3 + bl_ref[0, 0]
    logits_out[...] = _swish(t)
    z3 = jnp.sum(h, axis=0, keepdims=True)
    v = (jnp.dot(z1_ref[...], wv_ref[0:hh, :], preferred_element_type=jnp.float32)
         + jnp.dot(z2_ref[...], wv_ref[hh:2 * hh, :], preferred_element_type=jnp.float32)
         + jnp.dot(z3, wv_ref[2 * hh:3 * hh, :], preferred_element_type=jnp.float32)
         + bv_ref[0, 0])
    values_out[...] = _swish(v)


def _mp_body(n, e, h, ch, hf_hbm, src_hbm, dst_hbm, zero_hbm, out_hbm,
             shared_agg, src_v, dst_v, rows_v, sem):
    c = lax.axis_index("c")
    s = lax.axis_index("s")
    rows_per_sub = n // _NS

    pltpu.sync_copy(zero_hbm.at[pl.ds(s * rows_per_sub, rows_per_sub)],
                    shared_agg.at[pl.ds(s * rows_per_sub, rows_per_sub)])
    plsc.subcore_barrier()

    per_sub = e // (_NC * _NS)
    base = c * (e // _NC) + s * per_sub
    n_chunks = per_sub // ch

    def body(i, carry):
        off = base + i * ch
        pltpu.sync_copy(src_hbm.at[pl.ds(off, ch)], src_v)
        pltpu.sync_copy(dst_hbm.at[pl.ds(off, ch)], dst_v)
        pltpu.async_copy(hf_hbm.at[src_v], rows_v, sem).wait()
        pltpu.sync_copy(rows_v, shared_agg.at[dst_v], add=True)
        return carry

    lax.fori_loop(0, n_chunks, body, 0)
    plsc.subcore_barrier()

    pltpu.sync_copy(shared_agg.at[pl.ds(s * rows_per_sub, rows_per_sub)],
                    out_hbm.at[pl.ds(c * n + s * rows_per_sub, rows_per_sub)])


def _make_mp(n, e, h, ch):
    mesh = plsc.VectorSubcoreMesh(core_axis_name="c", subcore_axis_name="s")
    return pl.kernel(
        functools.partial(_mp_body, n, e, h, ch),
        out_type=jax.ShapeDtypeStruct((_NC * n, h), jnp.float32),
        mesh=mesh,
        scratch_types=[
            pltpu.VMEM_SHARED((n, h), jnp.float32),
            pltpu.VMEM((ch,), jnp.int32),
            pltpu.VMEM((ch,), jnp.int32),
            pltpu.VMEM((ch, h), jnp.float32),
            pltpu.SemaphoreType.DMA,
        ],
    )


def kernel(state, edge_index, emb, W0, b0, W1, b1, W2, b2, Wl, bl, Wv, bv):
    n = state.shape[0]
    e = edge_index.shape[1]
    h = emb.shape[1]

    src = edge_index[0].astype(jnp.int32)
    dst = edge_index[1].astype(jnp.int32)
    state2 = state.reshape(n, 1).astype(jnp.int32)
    zeros = jnp.zeros((n, h), jnp.float32)

    fdt = jnp.float32
    t0 = pl.pallas_call(
        _t0_body, out_shape=jax.ShapeDtypeStruct((n, h), fdt))
    tmid = pl.pallas_call(
        _tmid_body,
        out_shape=(jax.ShapeDtypeStruct((n, h), fdt),
                   jax.ShapeDtypeStruct((n, 1), fdt),
                   jax.ShapeDtypeStruct((1, h), fdt)))
    t3 = pl.pallas_call(
        _t3_body,
        out_shape=(jax.ShapeDtypeStruct((n, 1), fdt),
                   jax.ShapeDtypeStruct((1, 1), fdt)))

    mp = _make_mp(n, e, h, ch=80)

    hf1 = t0(state2, emb, W0, b0.reshape(1, h))
    agg1 = mp(hf1, src, dst, zeros)
    hf2, c1, z1 = tmid(hf1, agg1, W1, b1.reshape(1, h), Wl[0:h])
    agg2 = mp(hf2, src, dst, zeros)
    hf3, c2, z2 = tmid(hf2, agg2, W2, b2.reshape(1, h), Wl[h:2 * h])
    agg3 = mp(hf3, src, dst, zeros)
    logits2, values = t3(hf3, agg3, c1, c2, Wl[2 * h:3 * h],
                         bl.reshape(1, 1), z1, z2, Wv, bv.reshape(1, 1))
    return logits2.reshape(n), values


# speedup vs baseline: 4.9750x; 4.9750x over previous
import functools

import jax
import jax.numpy as jnp
from jax import lax
from jax.experimental import pallas as pl
from jax.experimental.pallas import tpu as pltpu
from jax.experimental.pallas import tpu_sc as plsc

_NC = 2
_NS = 16


def _swish(x):
    return x * jax.nn.sigmoid(x)


def _t0_body(state_ref, emb_ref, w_ref, b_ref, out_ref):
    st = state_ref[...]
    e0 = emb_ref[0:1, :]
    e1 = emb_ref[1:2, :]
    h = jnp.where(st == 0, e0, e1)
    x = jnp.dot(h, w_ref[...], preferred_element_type=jnp.float32) + b_ref[...]
    out_ref[...] = _swish(x)


def _tmid_body(hfp_ref, agg_ref, w_ref, b_ref, wl_ref, hf_out, c_out, z_out):
    n = hfp_ref.shape[0]
    npad = agg_ref.shape[0] // 2
    h = hfp_ref[...] + agg_ref[0:n, :] + agg_ref[npad:npad + n, :]
    x = jnp.dot(h, w_ref[...], preferred_element_type=jnp.float32) + b_ref[...]
    hf_out[...] = _swish(x)
    c_out[...] = jnp.dot(h, wl_ref[...], preferred_element_type=jnp.float32)
    z_out[...] = jnp.sum(h, axis=0, keepdims=True)


def _t3_body(hf3_ref, agg_ref, c1_ref, c2_ref, wl3_ref, bl_ref, z1_ref,
             z2_ref, wv_ref, bv_ref, logits_out, values_out):
    n = hf3_ref.shape[0]
    npad = agg_ref.shape[0] // 2
    hh = wv_ref.shape[0] // 3
    h = hf3_ref[...] + agg_ref[0:n, :] + agg_ref[npad:npad + n, :]
    c3 = jnp.dot(h, wl3_ref[...], preferred_element_type=jnp.float32)
    t = c1_ref[...] + c2_ref[...] + c3 + bl_ref[0, 0]
    logits_out[...] = _swish(t)
    z3 = jnp.sum(h, axis=0, keepdims=True)
    v = (jnp.dot(z1_ref[...], wv_ref[0:hh, :], preferred_element_type=jnp.float32)
         + jnp.dot(z2_ref[...], wv_ref[hh:2 * hh, :], preferred_element_type=jnp.float32)
         + jnp.dot(z3, wv_ref[2 * hh:3 * hh, :], preferred_element_type=jnp.float32)
         + bv_ref[0, 0])
    values_out[...] = _swish(v)


def _mp_body(n, e, h, ch, hf_hbm, src_hbm, dst_hbm, zero_hbm, out_hbm,
             shared_agg, src_v, dst_v, rows_v, sem):
    c = lax.axis_index("c")
    s = lax.axis_index("s")
    rows_per_sub = shared_agg.shape[0] // _NS

    pltpu.sync_copy(zero_hbm.at[pl.ds(s * rows_per_sub, rows_per_sub)],
                    shared_agg.at[pl.ds(s * rows_per_sub, rows_per_sub)])
    plsc.subcore_barrier()

    per_sub = e // (_NC * _NS)
    base = c * (e // _NC) + s * per_sub
    n_chunks = per_sub // ch

    def body(i, carry):
        off = base + i * ch
        pltpu.sync_copy(src_hbm.at[pl.ds(off, ch)], src_v)
        pltpu.sync_copy(dst_hbm.at[pl.ds(off, ch)], dst_v)
        pltpu.async_copy(hf_hbm.at[src_v], rows_v, sem).wait()
        pltpu.sync_copy(rows_v, shared_agg.at[dst_v], add=True)
        return carry

    lax.fori_loop(0, n_chunks, body, 0)
    plsc.subcore_barrier()

    npad = shared_agg.shape[0]
    pltpu.sync_copy(shared_agg.at[pl.ds(s * rows_per_sub, rows_per_sub)],
                    out_hbm.at[pl.ds(c * npad + s * rows_per_sub, rows_per_sub)])


def _round_up(x, m):
    return ((x + m - 1) // m) * m


def _make_mp(n, e, h, ch):
    npad = _round_up(-(-n // _NS), 8) * _NS
    mesh = plsc.VectorSubcoreMesh(core_axis_name="c", subcore_axis_name="s")
    return pl.kernel(
        functools.partial(_mp_body, n, e, h, ch),
        out_type=jax.ShapeDtypeStruct((_NC * npad, h), jnp.float32),
        mesh=mesh,
        scratch_types=[
            pltpu.VMEM_SHARED((npad, h), jnp.float32),
            pltpu.VMEM((ch,), jnp.int32),
            pltpu.VMEM((ch,), jnp.int32),
            pltpu.VMEM((ch, h), jnp.float32),
            pltpu.SemaphoreType.DMA,
        ],
    )


def kernel(state, edge_index, emb, W0, b0, W1, b1, W2, b2, Wl, bl, Wv, bv):
    n = state.shape[0]
    e = edge_index.shape[1]
    h = emb.shape[1]

    src = edge_index[0].astype(jnp.int32)
    dst = edge_index[1].astype(jnp.int32)
    state2 = state.reshape(n, 1).astype(jnp.int32)
    npad = _round_up(-(-n // _NS), 8) * _NS
    zeros = jnp.zeros((npad, h), jnp.float32)

    fdt = jnp.float32
    t0 = pl.pallas_call(
        _t0_body, out_shape=jax.ShapeDtypeStruct((n, h), fdt))
    tmid = pl.pallas_call(
        _tmid_body,
        out_shape=(jax.ShapeDtypeStruct((n, h), fdt),
                   jax.ShapeDtypeStruct((n, 1), fdt),
                   jax.ShapeDtypeStruct((1, h), fdt)))
    t3 = pl.pallas_call(
        _t3_body,
        out_shape=(jax.ShapeDtypeStruct((n, 1), fdt),
                   jax.ShapeDtypeStruct((1, 1), fdt)))

    mp = _make_mp(n, e, h, ch=80)

    hf1 = t0(state2, emb, W0, b0.reshape(1, h))
    agg1 = mp(hf1, src, dst, zeros)
    hf2, c1, z1 = tmid(hf1, agg1, W1, b1.reshape(1, h), Wl[0:h])
    agg2 = mp(hf2, src, dst, zeros)
    hf3, c2, z2 = tmid(hf2, agg2, W2, b2.reshape(1, h), Wl[h:2 * h])
    agg3 = mp(hf3, src, dst, zeros)
    logits2, values = t3(hf3, agg3, c1, c2, Wl[2 * h:3 * h],
                         bl.reshape(1, 1), z1, z2, Wv, bv.reshape(1, 1))
    return logits2.reshape(n), values
